# Initial kernel scaffold; baseline (speedup 1.0000x reference)
import jax
import jax.numpy as jnp
from jax.experimental import pallas as pl


def kernel(x, table):
    raise NotImplementedError("write your pallas kernel here")


# speedup vs baseline: 1.7267x; 1.7267x over previous
import jax
import jax.numpy as jnp
from jax.experimental import pallas as pl

BLK_S = 512


def _add_body(x_ref, t_ref, o_ref):
    o_ref[...] = x_ref[...] + t_ref[...][None, :, :]


def kernel(x, table):
    batch, seq, d = x.shape
    grid = (seq // BLK_S,)
    return pl.pallas_call(
        _add_body,
        grid=grid,
        in_specs=[
            pl.BlockSpec((batch, BLK_S, d), lambda i: (0, i, 0)),
            pl.BlockSpec((BLK_S, d), lambda i: (i, 0)),
        ],
        out_specs=pl.BlockSpec((batch, BLK_S, d), lambda i: (0, i, 0)),
        out_shape=jax.ShapeDtypeStruct((batch, seq, d), x.dtype),
    )(x, table)


# speedup vs baseline: 1.7441x; 1.0101x over previous
import jax
import jax.numpy as jnp
from jax.experimental import pallas as pl

BLK_S = 2048


def _add_body(x_ref, t_ref, o_ref):
    o_ref[...] = x_ref[...] + t_ref[...][None, :, :]


def kernel(x, table):
    batch, seq, d = x.shape
    grid = (seq // BLK_S, batch)
    return pl.pallas_call(
        _add_body,
        grid=grid,
        in_specs=[
            pl.BlockSpec((1, BLK_S, d), lambda i, j: (j, i, 0)),
            pl.BlockSpec((BLK_S, d), lambda i, j: (i, 0)),
        ],
        out_specs=pl.BlockSpec((1, BLK_S, d), lambda i, j: (j, i, 0)),
        out_shape=jax.ShapeDtypeStruct((batch, seq, d), x.dtype),
    )(x, table)
